# baseline (speedup 1.0000x reference)
import functools

import jax
import jax.numpy as jnp
from jax import lax
from jax.experimental import pallas as pl
from jax.experimental.pallas import tpu as pltpu
from jax.experimental.pallas import tpu_sc as plsc

N = 10000
E = 320000
G = 256
IN = 128
ED = 16
H = 256
FFN = 512
LAYERS = 5
MOLF = 32

NB = 1000
NBLK = N // NB
EBLK = 8000
_BN_EPS = 1e-5


def _dot16(a, b):
    return jnp.dot(a.astype(jnp.bfloat16), b.astype(jnp.bfloat16),
                   preferred_element_type=jnp.float32)


def _enc_body(x_ref, wt_ref, b_ref, out_ref):
    out_ref[...] = _dot16(x_ref[...], wt_ref[...]) + b_ref[0]


def _enc_split(x, wt, b, mb):
    m, din = x.shape
    wtq = jnp.concatenate([wt[:, 128 * q:128 * (q + 1)] for q in range(2)], 0)
    bq = jnp.concatenate(
        [b[:, 128 * q:128 * (q + 1)] for q in range(2)], 0).reshape(2, 1, 128)
    nb = m // mb
    return pl.pallas_call(
        _enc_body,
        grid=(nb, 2),
        in_specs=[
            pl.BlockSpec((mb, din), lambda i, j: (i, 0)),
            pl.BlockSpec((din, 128), lambda i, j: (j, 0)),
            pl.BlockSpec((1, 1, 128), lambda i, j: (j, 0, 0)),
        ],
        out_specs=pl.BlockSpec((mb, 128), lambda i, j: (j * nb + i, 0)),
        out_shape=jax.ShapeDtypeStruct((2 * m, 128), jnp.float32),
    )(x, wtq, bq)


def _acc_stats(i, st_ref, upd):
    @pl.when(i == 0)
    def _():
        st_ref[...] = upd

    @pl.when(i != 0)
    def _():
        st_ref[...] += upd


def _passA_body(eps_ref, h0_ref, h1_ref, a0_ref, a1_ref, wt_ref, b_ref,
                y_ref, st_ref):
    i = pl.program_id(0)
    eps1 = 1.0 + eps_ref[0]
    z = jnp.concatenate(
        [eps1 * h_ref[...] + a_ref[...]
         for h_ref, a_ref in ((h0_ref, a0_ref), (h1_ref, a1_ref))], axis=1)
    y = _dot16(z, wt_ref[...]) + b_ref[...]
    y_ref[...] = y
    ps = jnp.sum(y, axis=0, keepdims=True)
    upd = jnp.concatenate(
        [ps, jnp.zeros((7, y.shape[1]), jnp.float32)], axis=0)
    _acc_stats(i, st_ref, upd)


def _passA(eps, h4, agg4, wt, b):
    qspecs = [pl.BlockSpec((NB, 128),
                           (lambda q: (lambda i: (q * NBLK + i, 0)))(q))
              for q in range(2)]
    return pl.pallas_call(
        _passA_body,
        grid=(NBLK,),
        in_specs=(
            [pl.BlockSpec(memory_space=pltpu.SMEM)]
            + qspecs + qspecs
            + [pl.BlockSpec((H, 2 * H), lambda i: (0, 0)),
               pl.BlockSpec((1, 2 * H), lambda i: (0, 0))]
        ),
        out_specs=[
            pl.BlockSpec((NB, 2 * H), lambda i: (i, 0)),
            pl.BlockSpec((8, 2 * H), lambda i: (0, 0)),
        ],
        out_shape=[
            jax.ShapeDtypeStruct((N, 2 * H), jnp.float32),
            jax.ShapeDtypeStruct((8, 2 * H), jnp.float32),
        ],
    )(eps, h4, h4, agg4, agg4, wt, b)


def _sq_body(y_ref, sum_ref, sq_ref):
    i = pl.program_id(0)
    m = sum_ref[0:1, :] / jnp.float32(N)
    d = y_ref[...] - m
    ps = jnp.sum(d * d, axis=0, keepdims=True)
    upd = jnp.concatenate(
        [ps, jnp.zeros((7, d.shape[1]), jnp.float32)], axis=0)
    _acc_stats(i, sq_ref, upd)


def _sqstats(y, colsum):
    d = y.shape[1]
    return pl.pallas_call(
        _sq_body,
        grid=(NBLK,),
        in_specs=[
            pl.BlockSpec((NB, d), lambda i: (i, 0)),
            pl.BlockSpec((8, d), lambda i: (0, 0)),
        ],
        out_specs=pl.BlockSpec((8, d), lambda i: (0, 0)),
        out_shape=jax.ShapeDtypeStruct((8, d), jnp.float32),
    )(y, colsum)


def _passB_body(y1_ref, sm_ref, sq_ref, g_ref, bb_ref, wt_ref, b2_ref,
                y2_ref, st2_ref):
    i = pl.program_id(0)
    m = sm_ref[0:1, :] / jnp.float32(N)
    v = sq_ref[0:1, :] / jnp.float32(N)
    xn = (y1_ref[...] - m) / jnp.sqrt(v + _BN_EPS) * g_ref[...] + bb_ref[...]
    xr = jnp.maximum(xn, 0.0)
    y2 = _dot16(xr, wt_ref[...]) + b2_ref[...]
    y2_ref[...] = y2
    ps = jnp.sum(y2, axis=0, keepdims=True)
    upd = jnp.concatenate(
        [ps, jnp.zeros((7, y2.shape[1]), jnp.float32)], axis=0)
    _acc_stats(i, st2_ref, upd)


def _passB(y1, sm1, sq1, g, bb, wt, b2):
    return pl.pallas_call(
        _passB_body,
        grid=(NBLK,),
        in_specs=[
            pl.BlockSpec((NB, 2 * H), lambda i: (i, 0)),
            pl.BlockSpec((8, 2 * H), lambda i: (0, 0)),
            pl.BlockSpec((8, 2 * H), lambda i: (0, 0)),
            pl.BlockSpec((1, 2 * H), lambda i: (0, 0)),
            pl.BlockSpec((1, 2 * H), lambda i: (0, 0)),
            pl.BlockSpec((2 * H, H), lambda i: (0, 0)),
            pl.BlockSpec((1, H), lambda i: (0, 0)),
        ],
        out_specs=[
            pl.BlockSpec((NB, H), lambda i: (i, 0)),
            pl.BlockSpec((8, H), lambda i: (0, 0)),
        ],
        out_shape=[
            jax.ShapeDtypeStruct((N, H), jnp.float32),
            jax.ShapeDtypeStruct((8, H), jnp.float32),
        ],
    )(y1, sm1, sq1, g, bb, wt, b2)


def _passC_body(y2_ref, sm_ref, sq_ref, g_ref, bb_ref, h0_ref, h1_ref):
    m = sm_ref[0:1, :] / jnp.float32(N)
    v = sq_ref[0:1, :] / jnp.float32(N)
    xn = (y2_ref[...] - m) / jnp.sqrt(v + _BN_EPS) * g_ref[...] + bb_ref[...]
    h = jnp.maximum(xn, 0.0)
    for q, ref in enumerate((h0_ref, h1_ref)):
        ref[...] = h[:, 128 * q:128 * (q + 1)]


def _passC(y2, sm2, sq2, g, bb):
    hq = pl.pallas_call(
        _passC_body,
        grid=(NBLK,),
        in_specs=[
            pl.BlockSpec((NB, H), lambda i: (i, 0)),
            pl.BlockSpec((8, H), lambda i: (0, 0)),
            pl.BlockSpec((8, H), lambda i: (0, 0)),
            pl.BlockSpec((1, H), lambda i: (0, 0)),
            pl.BlockSpec((1, H), lambda i: (0, 0)),
        ],
        out_specs=[pl.BlockSpec((NB, 128), lambda i: (i, 0))] * 2,
        out_shape=[jax.ShapeDtypeStruct((N, 128), jnp.float32)] * 2,
    )(y2, sm2, sq2, g, bb)
    return jnp.concatenate(hq, axis=0)


def _pool_body(hcat_ref, batch_ref, pooled_ref, cnt_ref):
    i = pl.program_id(0)
    b = batch_ref[0, 0, :]
    oh = (b[None, :] == lax.broadcasted_iota(jnp.int32, (G, NB), 0)
          ).astype(jnp.float32)
    pp = jnp.dot(oh, hcat_ref[...], preferred_element_type=jnp.float32)
    cc = jnp.dot(oh, jnp.ones((NB, 128), jnp.float32),
                 preferred_element_type=jnp.float32)

    @pl.when(i == 0)
    def _():
        pooled_ref[...] = pp
        cnt_ref[...] = cc

    @pl.when(i != 0)
    def _():
        pooled_ref[...] += pp
        cnt_ref[...] += cc


def _pool(hcat, batch_r):
    hw = hcat.shape[1]
    return pl.pallas_call(
        _pool_body,
        grid=(NBLK,),
        in_specs=[
            pl.BlockSpec((NB, hw), lambda i: (i, 0)),
            pl.BlockSpec((1, 1, NB), lambda i: (i, 0, 0)),
        ],
        out_specs=[
            pl.BlockSpec((G, hw), lambda i: (0, 0)),
            pl.BlockSpec((G, 128), lambda i: (0, 0)),
        ],
        out_shape=[
            jax.ShapeDtypeStruct((G, hw), jnp.float32),
            jax.ShapeDtypeStruct((G, 128), jnp.float32),
        ],
    )(hcat, batch_r)


def _cls_body(pooled_ref, cnt_ref, mol_ref,
              w1_ref, b1_ref, g1_ref, bb1_ref,
              w2_ref, b2_ref, g2_ref, bb2_ref,
              wf_ref, bf_ref, out_ref):
    cnt = jnp.maximum(cnt_ref[:, 0:1], 1.0)
    xm = pooled_ref[...] / cnt
    xin = jnp.concatenate([xm, mol_ref[...]], axis=1)
    y = _dot16(xin, w1_ref[...]) + b1_ref[...]
    m = jnp.mean(y, axis=0, keepdims=True)
    d = y - m
    v = jnp.mean(d * d, axis=0, keepdims=True)
    y = jnp.maximum(d / jnp.sqrt(v + _BN_EPS) * g1_ref[...] + bb1_ref[...],
                    0.0)
    y2 = _dot16(y, w2_ref[...]) + b2_ref[...]
    m2 = jnp.mean(y2, axis=0, keepdims=True)
    d2 = y2 - m2
    v2 = jnp.mean(d2 * d2, axis=0, keepdims=True)
    y2 = jnp.maximum(
        d2 / jnp.sqrt(v2 + _BN_EPS) * g2_ref[...] + bb2_ref[...], 0.0)
    out_ref[...] = (
        _dot16(y2, wf_ref[...]) + bf_ref[...]
    )


def _cls(pooled, cnt, mol, w1, b1, g1, bb1, w2, b2, g2, bb2, wf, bf):
    hw = pooled.shape[1]
    return pl.pallas_call(
        _cls_body,
        out_shape=jax.ShapeDtypeStruct((G, 1), jnp.float32),
    )(pooled, cnt, mol, w1, b1, g1, bb1, w2, b2, g2, bb2, wf, bf)


_SC_CH = 160
_SC_ST = 80
_NSTR = _SC_CH // _SC_ST
_EPT = E // 16
_ACC_R = 5024
_SUPER = 3200


def _edge_kernel_body(h2, packed, ea2, agg, pbuf, srcv, dstv, rows, eav,
                      acc, sem):
    s = lax.axis_index("s")

    def superchunk(p, d, lo, pN, sc_i, nrows):
        pltpu.sync_copy(packed.at[s].at[pl.ds(sc_i * 8, nrows)],
                        pbuf.at[pl.ds(0, nrows)])
        for r in range(nrows):
            for k in range(25):
                f = r * 400 + k * 16
                v = pbuf[r, pl.ds(k * 16, 16)]
                sv = (v & 0x3FFF) + pN
                dr = jnp.right_shift(v, 14)
                bad = (dr < lo) | (dr >= lo + 5000)
                dv = jnp.where(bad, 5000, dr - lo)
                srcv[f // 80, pl.ds(f % 80, 16)] = sv
                dstv[f // 80, pl.ds(f % 80, 16)] = dv
        for t in range(nrows * 400 // _SC_CH):
            ea0 = p * E + s * _EPT + sc_i * _SUPER + t * _SC_CH
            pltpu.sync_copy(ea2.at[pl.ds(ea0, _SC_CH)], eav)
            handles = []
            for j in range(_NSTR):
                handles.append(pltpu.async_copy(
                    h2.at[srcv.at[t * _NSTR + j]],
                    rows.at[pl.ds(j * _SC_ST, _SC_ST)], sem))
            for hh in handles:
                hh.wait()

            def comp(r, _):
                for jj in range(8):
                    sl = pl.ds(jj * 16, 16)
                    eav[r, sl] = jnp.maximum(rows[r, sl] + eav[r, sl], 0.0)
                return 0

            lax.fori_loop(0, _SC_CH, comp, 0, unroll=2)
            for j in range(_NSTR):
                pltpu.sync_copy(eav.at[pl.ds(j * _SC_ST, _SC_ST)],
                                acc.at[dstv.at[t * _NSTR + j]], add=True)

    def phase(ph, carry):
        p = ph // 2
        d = ph - 2 * p
        lo = d * 5000
        pN = p * N

        def zfill(r, _):
            for jj in range(8):
                eav[r, pl.ds(jj * 16, 16)] = jnp.zeros((16,), jnp.float32)
            return 0

        lax.fori_loop(0, _SC_CH, zfill, 0, unroll=4)

        @pl.when(s < 15)
        def _():
            pltpu.sync_copy(eav.at[pl.ds(0, 160)],
                            acc.at[pl.ds(s * 320, 160)])
            pltpu.sync_copy(eav.at[pl.ds(0, 160)],
                            acc.at[pl.ds(s * 320 + 160, 160)])

        @pl.when(s == 15)
        def _():
            pltpu.sync_copy(eav.at[pl.ds(0, 160)],
                            acc.at[pl.ds(4800, 160)])
            pltpu.sync_copy(eav.at[pl.ds(0, _ACC_R - 4960)],
                            acc.at[pl.ds(4960, _ACC_R - 4960)])

        plsc.subcore_barrier()

        def sc_loop(sc_i, carry2):
            superchunk(p, d, lo, pN, sc_i, 8)
            return carry2

        lax.fori_loop(0, _EPT // _SUPER, sc_loop, 0)
        superchunk(p, d, lo, pN, _EPT // _SUPER, (_EPT % _SUPER) // 400)

        plsc.subcore_barrier()
        out0 = p * N + d * 5000
        pltpu.sync_copy(acc.at[pl.ds(s * 312, 312)],
                        agg.at[pl.ds(out0 + s * 312, 312)])

        @pl.when(s == 15)
        def _():
            pltpu.sync_copy(acc.at[pl.ds(4992, 8)],
                            agg.at[pl.ds(out0 + 4992, 8)])

        plsc.subcore_barrier()
        return carry

    lax.fori_loop(0, 4, phase, 0)


_edge_kernel_cache = []


def _edge_agg(h2, packed, ea2):
    if not _edge_kernel_cache:
        mesh = plsc.VectorSubcoreMesh(core_axis_name="c",
                                      subcore_axis_name="s", num_cores=1)
        _edge_kernel_cache.append(pl.kernel(
            _edge_kernel_body,
            mesh=mesh,
            out_type=jax.ShapeDtypeStruct((2 * N, 128), jnp.float32),
            scratch_types=[
                pltpu.VMEM((8, 400), jnp.int32),
                pltpu.VMEM((8 * 400 // _SC_ST, _SC_ST), jnp.int32),
                pltpu.VMEM((8 * 400 // _SC_ST, _SC_ST), jnp.int32),
                pltpu.VMEM((_SC_CH, 128), jnp.float32),
                pltpu.VMEM((_SC_CH, 128), jnp.float32),
                pltpu.VMEM_SHARED((_ACC_R, 128), jnp.float32),
                pltpu.SemaphoreType.DMA,
            ]))
    return _edge_kernel_cache[0](h2, packed, ea2)


def _row(v):
    return v.reshape(1, -1)


def kernel(x, edge_index, edge_attr, mol_features, batch, params):
    src = edge_index[0].astype(jnp.int32)
    dst = edge_index[1].astype(jnp.int32)
    packed = (src | (dst << 14)).reshape(16, _EPT // 400, 400)
    batch_r = batch.astype(jnp.int32).reshape(NBLK, 1, NB)

    pe = params["node_enc"]
    h2 = _enc_split(x, pe["w"].T, _row(pe["b"]), NB)
    pee = params["edge_enc"]
    ea2 = _enc_split(edge_attr, pee["w"].T, _row(pee["b"]), EBLK)

    convs = params["convs"]

    def _stk(f):
        return jnp.stack([f(c) for c in convs])

    xs = {
        "eps": _stk(lambda c: c["eps"].reshape(1)),
        "w1": _stk(lambda c: c["lin1"]["w"].T),
        "b1": _stk(lambda c: _row(c["lin1"]["b"])),
        "g1": _stk(lambda c: _row(c["bn1"]["g"])),
        "bb1": _stk(lambda c: _row(c["bn1"]["b"])),
        "w2": _stk(lambda c: c["lin2"]["w"].T),
        "b2": _stk(lambda c: _row(c["lin2"]["b"])),
        "g2": _stk(lambda c: _row(c["bn2"]["g"])),
        "bb2": _stk(lambda c: _row(c["bn2"]["b"])),
    }

    def _layer(h2c, xl):
        agg2 = _edge_agg(h2c, packed, ea2)
        y1, sm1 = _passA(xl["eps"], h2c, agg2, xl["w1"], xl["b1"])
        sq1 = _sqstats(y1, sm1)
        y2, sm2 = _passB(y1, sm1, sq1, xl["g1"], xl["bb1"], xl["w2"],
                         xl["b2"])
        sq2 = _sqstats(y2, sm2)
        h2n = _passC(y2, sm2, sq2, xl["g2"], xl["bb2"])
        return h2n, h2n

    _, reps_stack = lax.scan(_layer, h2, xs)
    reps = [h2] + [reps_stack[i] for i in range(LAYERS)]

    hcat = jnp.concatenate(
        [jnp.concatenate([r[:N], r[N:]], axis=1) for r in reps], axis=1)
    pooled, cnt = _pool(hcat, batch_r)

    c0, c1 = params["classifier"]
    fin = params["final"]
    out = _cls(pooled, cnt, mol_features,
               c0["lin"]["w"].T, _row(c0["lin"]["b"]),
               _row(c0["bn"]["g"]), _row(c0["bn"]["b"]),
               c1["lin"]["w"].T, _row(c1["lin"]["b"]),
               _row(c1["bn"]["g"]), _row(c1["bn"]["b"]),
               fin["w"].T, _row(fin["b"]))
    return out


# speedup vs baseline: 1.9283x; 1.9283x over previous
import functools

import jax
import jax.numpy as jnp
from jax import lax
from jax.experimental import pallas as pl
from jax.experimental.pallas import tpu as pltpu
from jax.experimental.pallas import tpu_sc as plsc

N = 10000
E = 320000
G = 256
IN = 128
ED = 16
H = 256
FFN = 512
LAYERS = 5
MOLF = 32

NB = 1000
NBLK = N // NB
EBLK = 8000
_BN_EPS = 1e-5


def _dot16(a, b):
    return jnp.dot(a.astype(jnp.bfloat16), b.astype(jnp.bfloat16),
                   preferred_element_type=jnp.float32)


def _enc_body(x_ref, wt_ref, b_ref, out_ref):
    out_ref[...] = _dot16(x_ref[...], wt_ref[...]) + b_ref[0]


def _enc_split(x, wt, b, mb):
    m, din = x.shape
    wtq = jnp.concatenate([wt[:, 128 * q:128 * (q + 1)] for q in range(2)], 0)
    bq = jnp.concatenate(
        [b[:, 128 * q:128 * (q + 1)] for q in range(2)], 0).reshape(2, 1, 128)
    nb = m // mb
    return pl.pallas_call(
        _enc_body,
        grid=(nb, 2),
        in_specs=[
            pl.BlockSpec((mb, din), lambda i, j: (i, 0)),
            pl.BlockSpec((din, 128), lambda i, j: (j, 0)),
            pl.BlockSpec((1, 1, 128), lambda i, j: (j, 0, 0)),
        ],
        out_specs=pl.BlockSpec((mb, 128), lambda i, j: (j * nb + i, 0)),
        out_shape=jax.ShapeDtypeStruct((2 * m, 128), jnp.float32),
    )(x, wtq, bq)


def _acc_stats(i, st_ref, upd):
    @pl.when(i == 0)
    def _():
        st_ref[...] = upd

    @pl.when(i != 0)
    def _():
        st_ref[...] += upd


def _passA_body(eps_ref, h0_ref, h1_ref, a0_ref, a1_ref, wt_ref, b_ref,
                y_ref, st_ref):
    i = pl.program_id(0)
    eps1 = 1.0 + eps_ref[0]
    z = jnp.concatenate(
        [eps1 * h_ref[...] + a_ref[...]
         for h_ref, a_ref in ((h0_ref, a0_ref), (h1_ref, a1_ref))], axis=1)
    y = _dot16(z, wt_ref[...]) + b_ref[...]
    y_ref[...] = y
    ps = jnp.sum(y, axis=0, keepdims=True)
    upd = jnp.concatenate(
        [ps, jnp.zeros((7, y.shape[1]), jnp.float32)], axis=0)
    _acc_stats(i, st_ref, upd)


def _passA(eps, h4, agg4, wt, b):
    qspecs = [pl.BlockSpec((NB, 128),
                           (lambda q: (lambda i: (q * NBLK + i, 0)))(q))
              for q in range(2)]
    return pl.pallas_call(
        _passA_body,
        grid=(NBLK,),
        in_specs=(
            [pl.BlockSpec(memory_space=pltpu.SMEM)]
            + qspecs + qspecs
            + [pl.BlockSpec((H, 2 * H), lambda i: (0, 0)),
               pl.BlockSpec((1, 2 * H), lambda i: (0, 0))]
        ),
        out_specs=[
            pl.BlockSpec((NB, 2 * H), lambda i: (i, 0)),
            pl.BlockSpec((8, 2 * H), lambda i: (0, 0)),
        ],
        out_shape=[
            jax.ShapeDtypeStruct((N, 2 * H), jnp.float32),
            jax.ShapeDtypeStruct((8, 2 * H), jnp.float32),
        ],
    )(eps, h4, h4, agg4, agg4, wt, b)


def _sq_body(y_ref, sum_ref, sq_ref):
    i = pl.program_id(0)
    m = sum_ref[0:1, :] / jnp.float32(N)
    d = y_ref[...] - m
    ps = jnp.sum(d * d, axis=0, keepdims=True)
    upd = jnp.concatenate(
        [ps, jnp.zeros((7, d.shape[1]), jnp.float32)], axis=0)
    _acc_stats(i, sq_ref, upd)


def _sqstats(y, colsum):
    d = y.shape[1]
    return pl.pallas_call(
        _sq_body,
        grid=(NBLK,),
        in_specs=[
            pl.BlockSpec((NB, d), lambda i: (i, 0)),
            pl.BlockSpec((8, d), lambda i: (0, 0)),
        ],
        out_specs=pl.BlockSpec((8, d), lambda i: (0, 0)),
        out_shape=jax.ShapeDtypeStruct((8, d), jnp.float32),
    )(y, colsum)


def _passB_body(y1_ref, sm_ref, sq_ref, g_ref, bb_ref, wt_ref, b2_ref,
                y2_ref, st2_ref):
    i = pl.program_id(0)
    m = sm_ref[0:1, :] / jnp.float32(N)
    v = sq_ref[0:1, :] / jnp.float32(N)
    xn = (y1_ref[...] - m) / jnp.sqrt(v + _BN_EPS) * g_ref[...] + bb_ref[...]
    xr = jnp.maximum(xn, 0.0)
    y2 = _dot16(xr, wt_ref[...]) + b2_ref[...]
    y2_ref[...] = y2
    ps = jnp.sum(y2, axis=0, keepdims=True)
    upd = jnp.concatenate(
        [ps, jnp.zeros((7, y2.shape[1]), jnp.float32)], axis=0)
    _acc_stats(i, st2_ref, upd)


def _passB(y1, sm1, sq1, g, bb, wt, b2):
    return pl.pallas_call(
        _passB_body,
        grid=(NBLK,),
        in_specs=[
            pl.BlockSpec((NB, 2 * H), lambda i: (i, 0)),
            pl.BlockSpec((8, 2 * H), lambda i: (0, 0)),
            pl.BlockSpec((8, 2 * H), lambda i: (0, 0)),
            pl.BlockSpec((1, 2 * H), lambda i: (0, 0)),
            pl.BlockSpec((1, 2 * H), lambda i: (0, 0)),
            pl.BlockSpec((2 * H, H), lambda i: (0, 0)),
            pl.BlockSpec((1, H), lambda i: (0, 0)),
        ],
        out_specs=[
            pl.BlockSpec((NB, H), lambda i: (i, 0)),
            pl.BlockSpec((8, H), lambda i: (0, 0)),
        ],
        out_shape=[
            jax.ShapeDtypeStruct((N, H), jnp.float32),
            jax.ShapeDtypeStruct((8, H), jnp.float32),
        ],
    )(y1, sm1, sq1, g, bb, wt, b2)


def _passC_body(y2_ref, sm_ref, sq_ref, g_ref, bb_ref, h0_ref, h1_ref):
    m = sm_ref[0:1, :] / jnp.float32(N)
    v = sq_ref[0:1, :] / jnp.float32(N)
    xn = (y2_ref[...] - m) / jnp.sqrt(v + _BN_EPS) * g_ref[...] + bb_ref[...]
    h = jnp.maximum(xn, 0.0)
    for q, ref in enumerate((h0_ref, h1_ref)):
        ref[...] = h[:, 128 * q:128 * (q + 1)]


def _passC(y2, sm2, sq2, g, bb):
    hq = pl.pallas_call(
        _passC_body,
        grid=(NBLK,),
        in_specs=[
            pl.BlockSpec((NB, H), lambda i: (i, 0)),
            pl.BlockSpec((8, H), lambda i: (0, 0)),
            pl.BlockSpec((8, H), lambda i: (0, 0)),
            pl.BlockSpec((1, H), lambda i: (0, 0)),
            pl.BlockSpec((1, H), lambda i: (0, 0)),
        ],
        out_specs=[pl.BlockSpec((NB, 128), lambda i: (i, 0))] * 2,
        out_shape=[jax.ShapeDtypeStruct((N, 128), jnp.float32)] * 2,
    )(y2, sm2, sq2, g, bb)
    return jnp.concatenate(hq, axis=0)


def _pool_body(hcat_ref, batch_ref, pooled_ref, cnt_ref):
    i = pl.program_id(0)
    b = batch_ref[0, 0, :]
    oh = (b[None, :] == lax.broadcasted_iota(jnp.int32, (G, NB), 0)
          ).astype(jnp.float32)
    pp = jnp.dot(oh, hcat_ref[...], preferred_element_type=jnp.float32)
    cc = jnp.dot(oh, jnp.ones((NB, 128), jnp.float32),
                 preferred_element_type=jnp.float32)

    @pl.when(i == 0)
    def _():
        pooled_ref[...] = pp
        cnt_ref[...] = cc

    @pl.when(i != 0)
    def _():
        pooled_ref[...] += pp
        cnt_ref[...] += cc


def _pool(hcat, batch_r):
    hw = hcat.shape[1]
    return pl.pallas_call(
        _pool_body,
        grid=(NBLK,),
        in_specs=[
            pl.BlockSpec((NB, hw), lambda i: (i, 0)),
            pl.BlockSpec((1, 1, NB), lambda i: (i, 0, 0)),
        ],
        out_specs=[
            pl.BlockSpec((G, hw), lambda i: (0, 0)),
            pl.BlockSpec((G, 128), lambda i: (0, 0)),
        ],
        out_shape=[
            jax.ShapeDtypeStruct((G, hw), jnp.float32),
            jax.ShapeDtypeStruct((G, 128), jnp.float32),
        ],
    )(hcat, batch_r)


def _cls_body(pooled_ref, cnt_ref, mol_ref,
              w1_ref, b1_ref, g1_ref, bb1_ref,
              w2_ref, b2_ref, g2_ref, bb2_ref,
              wf_ref, bf_ref, out_ref):
    cnt = jnp.maximum(cnt_ref[:, 0:1], 1.0)
    xm = pooled_ref[...] / cnt
    xin = jnp.concatenate([xm, mol_ref[...]], axis=1)
    y = _dot16(xin, w1_ref[...]) + b1_ref[...]
    m = jnp.mean(y, axis=0, keepdims=True)
    d = y - m
    v = jnp.mean(d * d, axis=0, keepdims=True)
    y = jnp.maximum(d / jnp.sqrt(v + _BN_EPS) * g1_ref[...] + bb1_ref[...],
                    0.0)
    y2 = _dot16(y, w2_ref[...]) + b2_ref[...]
    m2 = jnp.mean(y2, axis=0, keepdims=True)
    d2 = y2 - m2
    v2 = jnp.mean(d2 * d2, axis=0, keepdims=True)
    y2 = jnp.maximum(
        d2 / jnp.sqrt(v2 + _BN_EPS) * g2_ref[...] + bb2_ref[...], 0.0)
    out_ref[...] = (
        _dot16(y2, wf_ref[...]) + bf_ref[...]
    )


def _cls(pooled, cnt, mol, w1, b1, g1, bb1, w2, b2, g2, bb2, wf, bf):
    hw = pooled.shape[1]
    return pl.pallas_call(
        _cls_body,
        out_shape=jax.ShapeDtypeStruct((G, 1), jnp.float32),
    )(pooled, cnt, mol, w1, b1, g1, bb1, w2, b2, g2, bb2, wf, bf)


_SC_CH = 160
_SC_ST = 80
_NSTR = _SC_CH // _SC_ST
_EPT = E // 16
_SUPER = 3200
_HS = 800


def _edge_kernel_body(h2, packed, ea2, agg, pbuf, srcv, dstv, rows, eav,
                      acc, sem):
    s = lax.axis_index("s")

    def halfsuper(p, pN, sc_i, hs, nrows):
        for r in range(nrows):
            for k in range(25):
                f = r * 400 + k * 16
                v = pbuf[hs * 2 + r, pl.ds(k * 16, 16)]
                srcv[f // 80, pl.ds(f % 80, 16)] = (v & 0x3FFF) + pN
                dstv[f // 80, pl.ds(f % 80, 16)] = jnp.right_shift(v, 14)
        for t in range(nrows * 400 // _SC_CH):
            ea0 = (p * E + s * _EPT + sc_i * _SUPER + hs * _HS
                   + t * _SC_CH)
            pltpu.sync_copy(ea2.at[pl.ds(ea0, _SC_CH)], eav)
            handles = []
            for j in range(_NSTR):
                handles.append(pltpu.async_copy(
                    h2.at[srcv.at[t * _NSTR + j]],
                    rows.at[pl.ds(j * _SC_ST, _SC_ST)], sem))
            for hh in handles:
                hh.wait()

            def comp(r, _):
                for jj in range(8):
                    sl = pl.ds(jj * 16, 16)
                    eav[r, sl] = jnp.maximum(rows[r, sl] + eav[r, sl], 0.0)
                return 0

            lax.fori_loop(0, _SC_CH, comp, 0, unroll=2)
            for j in range(_NSTR):
                pltpu.sync_copy(eav.at[pl.ds(j * _SC_ST, _SC_ST)],
                                acc.at[dstv.at[t * _NSTR + j]], add=True)

    def phase(p, carry):
        pN = p * N

        def zfill(r, _):
            for jj in range(8):
                eav[r, pl.ds(jj * 16, 16)] = jnp.zeros((16,), jnp.float32)
            return 0

        lax.fori_loop(0, _SC_CH, zfill, 0, unroll=4)
        r0 = s * 624
        for off, ln in ((0, 160), (160, 160), (320, 160), (480, 144)):
            pltpu.sync_copy(eav.at[pl.ds(0, ln)],
                            acc.at[pl.ds(r0 + off, ln)])

        @pl.when(s == 15)
        def _():
            pltpu.sync_copy(eav.at[pl.ds(0, 16)], acc.at[pl.ds(9984, 16)])

        plsc.subcore_barrier()

        def sc_loop(sc_i, carry2):
            pltpu.sync_copy(packed.at[s].at[pl.ds(sc_i * 8, 8)], pbuf)
            for hs in range(4):
                halfsuper(p, pN, sc_i, hs, 2)
            return carry2

        lax.fori_loop(0, _EPT // _SUPER, sc_loop, 0)
        pltpu.sync_copy(
            packed.at[s].at[pl.ds((_EPT // _SUPER) * 8, 2)],
            pbuf.at[pl.ds(0, 2)])
        halfsuper(p, pN, _EPT // _SUPER, 0, 2)

        plsc.subcore_barrier()
        pltpu.sync_copy(acc.at[pl.ds(r0, 624)],
                        agg.at[pl.ds(p * N + r0, 624)])

        @pl.when(s == 15)
        def _():
            pltpu.sync_copy(acc.at[pl.ds(9984, 16)],
                            agg.at[pl.ds(p * N + 9984, 16)])

        plsc.subcore_barrier()
        return carry

    lax.fori_loop(0, 2, phase, 0)


_edge_kernel_cache = []


def _edge_agg(h2, packed, ea2):
    if not _edge_kernel_cache:
        mesh = plsc.VectorSubcoreMesh(core_axis_name="c",
                                      subcore_axis_name="s", num_cores=1)
        _edge_kernel_cache.append(pl.kernel(
            _edge_kernel_body,
            mesh=mesh,
            out_type=jax.ShapeDtypeStruct((2 * N, 128), jnp.float32),
            scratch_types=[
                pltpu.VMEM((8, 400), jnp.int32),
                pltpu.VMEM((_HS // _SC_ST, _SC_ST), jnp.int32),
                pltpu.VMEM((_HS // _SC_ST, _SC_ST), jnp.int32),
                pltpu.VMEM((_SC_CH, 128), jnp.float32),
                pltpu.VMEM((_SC_CH, 128), jnp.float32),
                pltpu.VMEM_SHARED((N, 128), jnp.float32),
                pltpu.SemaphoreType.DMA,
            ]))
    return _edge_kernel_cache[0](h2, packed, ea2)


def _row(v):
    return v.reshape(1, -1)


def kernel(x, edge_index, edge_attr, mol_features, batch, params):
    src = edge_index[0].astype(jnp.int32)
    dst = edge_index[1].astype(jnp.int32)
    packed = (src | (dst << 14)).reshape(16, _EPT // 400, 400)
    batch_r = batch.astype(jnp.int32).reshape(NBLK, 1, NB)

    pe = params["node_enc"]
    h2 = _enc_split(x, pe["w"].T, _row(pe["b"]), NB)
    pee = params["edge_enc"]
    ea2 = _enc_split(edge_attr, pee["w"].T, _row(pee["b"]), EBLK)

    convs = params["convs"]

    def _stk(f):
        return jnp.stack([f(c) for c in convs])

    xs = {
        "eps": _stk(lambda c: c["eps"].reshape(1)),
        "w1": _stk(lambda c: c["lin1"]["w"].T),
        "b1": _stk(lambda c: _row(c["lin1"]["b"])),
        "g1": _stk(lambda c: _row(c["bn1"]["g"])),
        "bb1": _stk(lambda c: _row(c["bn1"]["b"])),
        "w2": _stk(lambda c: c["lin2"]["w"].T),
        "b2": _stk(lambda c: _row(c["lin2"]["b"])),
        "g2": _stk(lambda c: _row(c["bn2"]["g"])),
        "bb2": _stk(lambda c: _row(c["bn2"]["b"])),
    }

    def _layer(h2c, xl):
        agg2 = _edge_agg(h2c, packed, ea2)
        y1, sm1 = _passA(xl["eps"], h2c, agg2, xl["w1"], xl["b1"])
        sq1 = _sqstats(y1, sm1)
        y2, sm2 = _passB(y1, sm1, sq1, xl["g1"], xl["bb1"], xl["w2"],
                         xl["b2"])
        sq2 = _sqstats(y2, sm2)
        h2n = _passC(y2, sm2, sq2, xl["g2"], xl["bb2"])
        return h2n, h2n

    _, reps_stack = lax.scan(_layer, h2, xs)
    reps = [h2] + [reps_stack[i] for i in range(LAYERS)]

    hcat = jnp.concatenate(
        [jnp.concatenate([r[:N], r[N:]], axis=1) for r in reps], axis=1)
    pooled, cnt = _pool(hcat, batch_r)

    c0, c1 = params["classifier"]
    fin = params["final"]
    out = _cls(pooled, cnt, mol_features,
               c0["lin"]["w"].T, _row(c0["lin"]["b"]),
               _row(c0["bn"]["g"]), _row(c0["bn"]["b"]),
               c1["lin"]["w"].T, _row(c1["lin"]["b"]),
               _row(c1["bn"]["g"]), _row(c1["bn"]["b"]),
               fin["w"].T, _row(fin["b"]))
    return out


# speedup vs baseline: 2.4728x; 1.2824x over previous
import functools

import jax
import jax.numpy as jnp
from jax import lax
from jax.experimental import pallas as pl
from jax.experimental.pallas import tpu as pltpu
from jax.experimental.pallas import tpu_sc as plsc

N = 10000
E = 320000
G = 256
IN = 128
ED = 16
H = 256
FFN = 512
LAYERS = 5
MOLF = 32

NB = 1000
NBLK = N // NB
EBLK = 8000
_BN_EPS = 1e-5


def _dot16(a, b):
    return jnp.dot(a.astype(jnp.bfloat16), b.astype(jnp.bfloat16),
                   preferred_element_type=jnp.float32)


def _enc_body(x_ref, wt_ref, b_ref, out_ref):
    out_ref[...] = _dot16(x_ref[...], wt_ref[...]) + b_ref[0]


def _enc_split(x, wt, b, mb):
    m, din = x.shape
    wtq = jnp.concatenate([wt[:, 128 * q:128 * (q + 1)] for q in range(2)], 0)
    bq = jnp.concatenate(
        [b[:, 128 * q:128 * (q + 1)] for q in range(2)], 0).reshape(2, 1, 128)
    nb = m // mb
    return pl.pallas_call(
        _enc_body,
        grid=(nb, 2),
        in_specs=[
            pl.BlockSpec((mb, din), lambda i, j: (i, 0)),
            pl.BlockSpec((din, 128), lambda i, j: (j, 0)),
            pl.BlockSpec((1, 1, 128), lambda i, j: (j, 0, 0)),
        ],
        out_specs=pl.BlockSpec((mb, 128), lambda i, j: (j * nb + i, 0)),
        out_shape=jax.ShapeDtypeStruct((2 * m, 128), jnp.float32),
    )(x, wtq, bq)


def _acc_stats(i, st_ref, upd):
    @pl.when(i == 0)
    def _():
        st_ref[...] = upd

    @pl.when(i != 0)
    def _():
        st_ref[...] += upd


def _passA_body(eps_ref, h0_ref, h1_ref, a0_ref, a1_ref, wt_ref, b_ref,
                y_ref, st_ref):
    i = pl.program_id(0)
    eps1 = 1.0 + eps_ref[0]
    z = jnp.concatenate(
        [eps1 * h_ref[...] + a_ref[...]
         for h_ref, a_ref in ((h0_ref, a0_ref), (h1_ref, a1_ref))], axis=1)
    y = _dot16(z, wt_ref[...]) + b_ref[...]
    y_ref[...] = y
    ps = jnp.sum(y, axis=0, keepdims=True)
    upd = jnp.concatenate(
        [ps, jnp.zeros((7, y.shape[1]), jnp.float32)], axis=0)
    _acc_stats(i, st_ref, upd)


def _passA(eps, h4, agg4, wt, b):
    qspecs = [pl.BlockSpec((NB, 128),
                           (lambda q: (lambda i: (q * NBLK + i, 0)))(q))
              for q in range(2)]
    return pl.pallas_call(
        _passA_body,
        grid=(NBLK,),
        in_specs=(
            [pl.BlockSpec(memory_space=pltpu.SMEM)]
            + qspecs + qspecs
            + [pl.BlockSpec((H, 2 * H), lambda i: (0, 0)),
               pl.BlockSpec((1, 2 * H), lambda i: (0, 0))]
        ),
        out_specs=[
            pl.BlockSpec((NB, 2 * H), lambda i: (i, 0)),
            pl.BlockSpec((8, 2 * H), lambda i: (0, 0)),
        ],
        out_shape=[
            jax.ShapeDtypeStruct((N, 2 * H), jnp.float32),
            jax.ShapeDtypeStruct((8, 2 * H), jnp.float32),
        ],
    )(eps, h4, h4, agg4, agg4, wt, b)


def _sq_body(y_ref, sum_ref, sq_ref):
    i = pl.program_id(0)
    m = sum_ref[0:1, :] / jnp.float32(N)
    d = y_ref[...] - m
    ps = jnp.sum(d * d, axis=0, keepdims=True)
    upd = jnp.concatenate(
        [ps, jnp.zeros((7, d.shape[1]), jnp.float32)], axis=0)
    _acc_stats(i, sq_ref, upd)


def _sqstats(y, colsum):
    d = y.shape[1]
    return pl.pallas_call(
        _sq_body,
        grid=(NBLK,),
        in_specs=[
            pl.BlockSpec((NB, d), lambda i: (i, 0)),
            pl.BlockSpec((8, d), lambda i: (0, 0)),
        ],
        out_specs=pl.BlockSpec((8, d), lambda i: (0, 0)),
        out_shape=jax.ShapeDtypeStruct((8, d), jnp.float32),
    )(y, colsum)


def _passB_body(y1_ref, sm_ref, sq_ref, g_ref, bb_ref, wt_ref, b2_ref,
                y2_ref, st2_ref):
    i = pl.program_id(0)
    m = sm_ref[0:1, :] / jnp.float32(N)
    v = sq_ref[0:1, :] / jnp.float32(N)
    xn = (y1_ref[...] - m) / jnp.sqrt(v + _BN_EPS) * g_ref[...] + bb_ref[...]
    xr = jnp.maximum(xn, 0.0)
    y2 = _dot16(xr, wt_ref[...]) + b2_ref[...]
    y2_ref[...] = y2
    ps = jnp.sum(y2, axis=0, keepdims=True)
    upd = jnp.concatenate(
        [ps, jnp.zeros((7, y2.shape[1]), jnp.float32)], axis=0)
    _acc_stats(i, st2_ref, upd)


def _passB(y1, sm1, sq1, g, bb, wt, b2):
    return pl.pallas_call(
        _passB_body,
        grid=(NBLK,),
        in_specs=[
            pl.BlockSpec((NB, 2 * H), lambda i: (i, 0)),
            pl.BlockSpec((8, 2 * H), lambda i: (0, 0)),
            pl.BlockSpec((8, 2 * H), lambda i: (0, 0)),
            pl.BlockSpec((1, 2 * H), lambda i: (0, 0)),
            pl.BlockSpec((1, 2 * H), lambda i: (0, 0)),
            pl.BlockSpec((2 * H, H), lambda i: (0, 0)),
            pl.BlockSpec((1, H), lambda i: (0, 0)),
        ],
        out_specs=[
            pl.BlockSpec((NB, H), lambda i: (i, 0)),
            pl.BlockSpec((8, H), lambda i: (0, 0)),
        ],
        out_shape=[
            jax.ShapeDtypeStruct((N, H), jnp.float32),
            jax.ShapeDtypeStruct((8, H), jnp.float32),
        ],
    )(y1, sm1, sq1, g, bb, wt, b2)


def _passC_body(y2_ref, sm_ref, sq_ref, g_ref, bb_ref, h0_ref, h1_ref):
    m = sm_ref[0:1, :] / jnp.float32(N)
    v = sq_ref[0:1, :] / jnp.float32(N)
    xn = (y2_ref[...] - m) / jnp.sqrt(v + _BN_EPS) * g_ref[...] + bb_ref[...]
    h = jnp.maximum(xn, 0.0)
    for q, ref in enumerate((h0_ref, h1_ref)):
        ref[...] = h[:, 128 * q:128 * (q + 1)]


def _passC(y2, sm2, sq2, g, bb):
    hq = pl.pallas_call(
        _passC_body,
        grid=(NBLK,),
        in_specs=[
            pl.BlockSpec((NB, H), lambda i: (i, 0)),
            pl.BlockSpec((8, H), lambda i: (0, 0)),
            pl.BlockSpec((8, H), lambda i: (0, 0)),
            pl.BlockSpec((1, H), lambda i: (0, 0)),
            pl.BlockSpec((1, H), lambda i: (0, 0)),
        ],
        out_specs=[pl.BlockSpec((NB, 128), lambda i: (i, 0))] * 2,
        out_shape=[jax.ShapeDtypeStruct((N, 128), jnp.float32)] * 2,
    )(y2, sm2, sq2, g, bb)
    return jnp.concatenate(hq, axis=0)


def _pool_body(hcat_ref, batch_ref, pooled_ref, cnt_ref):
    i = pl.program_id(0)
    b = batch_ref[0, 0, :]
    oh = (b[None, :] == lax.broadcasted_iota(jnp.int32, (G, NB), 0)
          ).astype(jnp.float32)
    pp = jnp.dot(oh, hcat_ref[...], preferred_element_type=jnp.float32)
    cc = jnp.dot(oh, jnp.ones((NB, 128), jnp.float32),
                 preferred_element_type=jnp.float32)

    @pl.when(i == 0)
    def _():
        pooled_ref[...] = pp
        cnt_ref[...] = cc

    @pl.when(i != 0)
    def _():
        pooled_ref[...] += pp
        cnt_ref[...] += cc


def _pool(hcat, batch_r):
    hw = hcat.shape[1]
    return pl.pallas_call(
        _pool_body,
        grid=(NBLK,),
        in_specs=[
            pl.BlockSpec((NB, hw), lambda i: (i, 0)),
            pl.BlockSpec((1, 1, NB), lambda i: (i, 0, 0)),
        ],
        out_specs=[
            pl.BlockSpec((G, hw), lambda i: (0, 0)),
            pl.BlockSpec((G, 128), lambda i: (0, 0)),
        ],
        out_shape=[
            jax.ShapeDtypeStruct((G, hw), jnp.float32),
            jax.ShapeDtypeStruct((G, 128), jnp.float32),
        ],
    )(hcat, batch_r)


def _cls_body(pooled_ref, cnt_ref, mol_ref,
              w1_ref, b1_ref, g1_ref, bb1_ref,
              w2_ref, b2_ref, g2_ref, bb2_ref,
              wf_ref, bf_ref, out_ref):
    cnt = jnp.maximum(cnt_ref[:, 0:1], 1.0)
    xm = pooled_ref[...] / cnt
    xin = jnp.concatenate([xm, mol_ref[...]], axis=1)
    y = _dot16(xin, w1_ref[...]) + b1_ref[...]
    m = jnp.mean(y, axis=0, keepdims=True)
    d = y - m
    v = jnp.mean(d * d, axis=0, keepdims=True)
    y = jnp.maximum(d / jnp.sqrt(v + _BN_EPS) * g1_ref[...] + bb1_ref[...],
                    0.0)
    y2 = _dot16(y, w2_ref[...]) + b2_ref[...]
    m2 = jnp.mean(y2, axis=0, keepdims=True)
    d2 = y2 - m2
    v2 = jnp.mean(d2 * d2, axis=0, keepdims=True)
    y2 = jnp.maximum(
        d2 / jnp.sqrt(v2 + _BN_EPS) * g2_ref[...] + bb2_ref[...], 0.0)
    out_ref[...] = (
        _dot16(y2, wf_ref[...]) + bf_ref[...]
    )


def _cls(pooled, cnt, mol, w1, b1, g1, bb1, w2, b2, g2, bb2, wf, bf):
    hw = pooled.shape[1]
    return pl.pallas_call(
        _cls_body,
        out_shape=jax.ShapeDtypeStruct((G, 1), jnp.float32),
    )(pooled, cnt, mol, w1, b1, g1, bb1, w2, b2, g2, bb2, wf, bf)


_SC_ST = 80
_EPT = E // 16
_SUPER = 3200
_HS = 800
_NSUB = _HS // _SC_ST


def _edge_kernel_body(h2, packed, ea2, agg, pbuf, srcv, dstv,
                      rows0, rows1, eav0, eav1, acc,
                      semg0, semg1, seme0, seme1, sems0, sems1):
    s = lax.axis_index("s")
    bufs = ((rows0, eav0, semg0, seme0, sems0),
            (rows1, eav1, semg1, seme1, sems1))

    def halfsuper(p, pN, ea_base, hs):
        for r in range(2):
            for k in range(25):
                f = r * 400 + k * 16
                v = pbuf[hs * 2 + r, pl.ds(k * 16, 16)]
                srcv[f // 80, pl.ds(f % 80, 16)] = (v & 0x3FFF) + pN
                dstv[f // 80, pl.ds(f % 80, 16)] = jnp.right_shift(v, 14)

        ea0 = ea_base + hs * _HS
        scat = [None, None]

        def issue_loads(t):
            rws, eav, sg, se, _ = bufs[t % 2]
            ge = pltpu.async_copy(ea2.at[pl.ds(ea0 + t * _SC_ST, _SC_ST)],
                                  eav, se)
            gh = pltpu.async_copy(h2.at[srcv.at[t]], rws, sg)
            return ge, gh

        pend = issue_loads(0)
        for t in range(_NSUB):
            rws, eav, sg, se, ss = bufs[t % 2]
            ge, gh = pend
            if t + 1 < _NSUB:
                if scat[(t + 1) % 2] is not None:
                    scat[(t + 1) % 2].wait()
                    scat[(t + 1) % 2] = None
                pend = issue_loads(t + 1)
            ge.wait()
            gh.wait()

            def comp(r, _):
                for jj in range(8):
                    sl = pl.ds(jj * 16, 16)
                    eav[r, sl] = jnp.maximum(rws[r, sl] + eav[r, sl], 0.0)
                return 0

            lax.fori_loop(0, _SC_ST, comp, 0, unroll=4)
            scat[t % 2] = pltpu.async_copy(eav, acc.at[dstv.at[t]], ss,
                                           add=True)
        for w in scat:
            if w is not None:
                w.wait()

    def phase(p, carry):
        pN = p * N

        def zfill(r, _):
            for jj in range(8):
                eav0[r, pl.ds(jj * 16, 16)] = jnp.zeros((16,), jnp.float32)
            return 0

        lax.fori_loop(0, _SC_ST, zfill, 0, unroll=4)
        r0 = s * 624
        for off, ln in ((0, 80), (80, 80), (160, 80), (240, 80),
                        (320, 80), (400, 80), (480, 80), (560, 64)):
            pltpu.sync_copy(eav0.at[pl.ds(0, ln)],
                            acc.at[pl.ds(r0 + off, ln)])

        @pl.when(s == 15)
        def _():
            pltpu.sync_copy(eav0.at[pl.ds(0, 16)], acc.at[pl.ds(9984, 16)])

        plsc.subcore_barrier()
        ea_p = p * E + s * _EPT

        def sc_loop(sc_i, carry2):
            pltpu.sync_copy(packed.at[s].at[pl.ds(sc_i * 8, 8)], pbuf)

            def hs_loop(hs, carry3):
                halfsuper(p, pN, ea_p + sc_i * _SUPER, hs)
                return carry3

            lax.fori_loop(0, 4, hs_loop, 0)
            return carry2

        lax.fori_loop(0, _EPT // _SUPER, sc_loop, 0)
        pltpu.sync_copy(packed.at[s].at[pl.ds((_EPT // _SUPER) * 8, 2)],
                        pbuf.at[pl.ds(0, 2)])
        halfsuper(p, pN, ea_p + (_EPT // _SUPER) * _SUPER, 0)

        plsc.subcore_barrier()
        pltpu.sync_copy(acc.at[pl.ds(r0, 624)],
                        agg.at[pl.ds(p * N + r0, 624)])

        @pl.when(s == 15)
        def _():
            pltpu.sync_copy(acc.at[pl.ds(9984, 16)],
                            agg.at[pl.ds(p * N + 9984, 16)])

        plsc.subcore_barrier()
        return carry

    lax.fori_loop(0, 2, phase, 0)


_edge_kernel_cache = []


def _edge_agg(h2, packed, ea2):
    if not _edge_kernel_cache:
        mesh = plsc.VectorSubcoreMesh(core_axis_name="c",
                                      subcore_axis_name="s", num_cores=1)
        _edge_kernel_cache.append(pl.kernel(
            _edge_kernel_body,
            mesh=mesh,
            out_type=jax.ShapeDtypeStruct((2 * N, 128), jnp.float32),
            scratch_types=[
                pltpu.VMEM((8, 400), jnp.int32),
                pltpu.VMEM((_NSUB, _SC_ST), jnp.int32),
                pltpu.VMEM((_NSUB, _SC_ST), jnp.int32),
                pltpu.VMEM((_SC_ST, 128), jnp.float32),
                pltpu.VMEM((_SC_ST, 128), jnp.float32),
                pltpu.VMEM((_SC_ST, 128), jnp.float32),
                pltpu.VMEM((_SC_ST, 128), jnp.float32),
                pltpu.VMEM_SHARED((N, 128), jnp.float32),
                pltpu.SemaphoreType.DMA,
                pltpu.SemaphoreType.DMA,
                pltpu.SemaphoreType.DMA,
                pltpu.SemaphoreType.DMA,
                pltpu.SemaphoreType.DMA,
                pltpu.SemaphoreType.DMA,
            ]))
    return _edge_kernel_cache[0](h2, packed, ea2)


def _row(v):
    return v.reshape(1, -1)


def kernel(x, edge_index, edge_attr, mol_features, batch, params):
    src = edge_index[0].astype(jnp.int32)
    dst = edge_index[1].astype(jnp.int32)
    packed = (src | (dst << 14)).reshape(16, _EPT // 400, 400)
    batch_r = batch.astype(jnp.int32).reshape(NBLK, 1, NB)

    pe = params["node_enc"]
    h2 = _enc_split(x, pe["w"].T, _row(pe["b"]), NB)
    pee = params["edge_enc"]
    ea2 = _enc_split(edge_attr, pee["w"].T, _row(pee["b"]), EBLK)

    convs = params["convs"]

    def _stk(f):
        return jnp.stack([f(c) for c in convs])

    xs = {
        "eps": _stk(lambda c: c["eps"].reshape(1)),
        "w1": _stk(lambda c: c["lin1"]["w"].T),
        "b1": _stk(lambda c: _row(c["lin1"]["b"])),
        "g1": _stk(lambda c: _row(c["bn1"]["g"])),
        "bb1": _stk(lambda c: _row(c["bn1"]["b"])),
        "w2": _stk(lambda c: c["lin2"]["w"].T),
        "b2": _stk(lambda c: _row(c["lin2"]["b"])),
        "g2": _stk(lambda c: _row(c["bn2"]["g"])),
        "bb2": _stk(lambda c: _row(c["bn2"]["b"])),
    }

    def _layer(h2c, xl):
        agg2 = _edge_agg(h2c, packed, ea2)
        y1, sm1 = _passA(xl["eps"], h2c, agg2, xl["w1"], xl["b1"])
        sq1 = _sqstats(y1, sm1)
        y2, sm2 = _passB(y1, sm1, sq1, xl["g1"], xl["bb1"], xl["w2"],
                         xl["b2"])
        sq2 = _sqstats(y2, sm2)
        h2n = _passC(y2, sm2, sq2, xl["g2"], xl["bb2"])
        return h2n, h2n

    _, reps_stack = lax.scan(_layer, h2, xs)
    reps = [h2] + [reps_stack[i] for i in range(LAYERS)]

    hcat = jnp.concatenate(
        [jnp.concatenate([r[:N], r[N:]], axis=1) for r in reps], axis=1)
    pooled, cnt = _pool(hcat, batch_r)

    c0, c1 = params["classifier"]
    fin = params["final"]
    out = _cls(pooled, cnt, mol_features,
               c0["lin"]["w"].T, _row(c0["lin"]["b"]),
               _row(c0["bn"]["g"]), _row(c0["bn"]["b"]),
               c1["lin"]["w"].T, _row(c1["lin"]["b"]),
               _row(c1["bn"]["g"]), _row(c1["bn"]["b"]),
               fin["w"].T, _row(fin["b"]))
    return out
